# baseline (speedup 1.0000x reference)
import functools
import math

import jax
import jax.numpy as jnp
from jax.experimental import pallas as pl
from jax.experimental.pallas import tpu as pltpu


def _moe_kernel(x_ref, rw_ref, w1_ref, b1_ref, w2_ref, b2_ref, out_ref, w_ref,
                *, n_ffn_tiles, expert_col, precision):
    f = pl.program_id(1)

    @pl.when(f == 0)
    def _router():
        scores = jax.lax.dot_general(
            x_ref[...], rw_ref[...], (((1,), (1,)), ((), ())),
            precision=jax.lax.Precision.HIGHEST,
            preferred_element_type=jnp.float32)
        m = jnp.max(scores, axis=1, keepdims=True)
        p = jnp.exp(scores - m)
        denom = jnp.sum(p, axis=1, keepdims=True)
        w_ref[...] = p[:, expert_col:expert_col + 1] / denom

    h = jax.lax.dot_general(
        x_ref[...], w1_ref[...], (((1,), (1,)), ((), ())),
        precision=precision, preferred_element_type=jnp.float32)
    h = h + b1_ref[...]
    h = 0.5 * h * (1.0 + jax.lax.erf(h * (1.0 / math.sqrt(2.0))))
    part = jax.lax.dot_general(
        h, w2_ref[...], (((1,), (1,)), ((), ())),
        precision=precision, preferred_element_type=jnp.float32)

    @pl.when(f == 0)
    def _init():
        out_ref[...] = part

    @pl.when(f != 0)
    def _acc():
        out_ref[...] += part

    @pl.when(f == n_ffn_tiles - 1)
    def _finish():
        out_ref[...] = (out_ref[...] + b2_ref[...]) * w_ref[...]


@functools.partial(jax.jit, static_argnames=())
def kernel(x, router_w, W1, b1, W2, b2):
    B_, S_, H_ = x.shape
    E_, FFN_, _ = W1.shape
    T = B_ * S_
    x_flat = x.reshape(T, H_)
    eid = E_ - 1
    w1 = W1[eid]
    b1e = b1[eid].reshape(1, FFN_)
    w2 = W2[eid]
    b2e = b2[eid].reshape(1, H_)

    TM = 512
    FK = 1024
    n_t = T // TM
    n_f = FFN_ // FK

    out = pl.pallas_call(
        functools.partial(_moe_kernel, n_ffn_tiles=n_f, expert_col=eid,
                          precision=jax.lax.Precision.HIGHEST),
        grid=(n_t, n_f),
        in_specs=[
            pl.BlockSpec((TM, H_), lambda t, f: (t, 0)),
            pl.BlockSpec((E_, H_), lambda t, f: (0, 0)),
            pl.BlockSpec((FK, H_), lambda t, f: (f, 0)),
            pl.BlockSpec((1, FK), lambda t, f: (0, f)),
            pl.BlockSpec((H_, FK), lambda t, f: (0, f)),
            pl.BlockSpec((1, H_), lambda t, f: (0, 0)),
        ],
        out_specs=pl.BlockSpec((TM, H_), lambda t, f: (t, 0)),
        out_shape=jax.ShapeDtypeStruct((T, H_), jnp.float32),
        scratch_shapes=[pltpu.VMEM((TM, 1), jnp.float32)],
        compiler_params=pltpu.CompilerParams(
            dimension_semantics=("parallel", "arbitrary")),
    )(x_flat, router_w, w1, b1e, w2, b2e)
    return out.reshape(B_, S_, H_)


# speedup vs baseline: 2.8578x; 2.8578x over previous
import functools
import math

import jax
import jax.numpy as jnp
from jax.experimental import pallas as pl
from jax.experimental.pallas import tpu as pltpu


def _moe_kernel(x_ref, rw_ref, w1_ref, b1_ref, w2_ref, b2_ref, out_ref, w_ref,
                *, n_ffn_tiles, expert_col, precision):
    f = pl.program_id(1)

    @pl.when(f == 0)
    def _router():
        scores = jax.lax.dot_general(
            x_ref[...], rw_ref[...], (((1,), (1,)), ((), ())),
            precision=jax.lax.Precision.HIGHEST,
            preferred_element_type=jnp.float32)
        m = jnp.max(scores, axis=1, keepdims=True)
        p = jnp.exp(scores - m)
        denom = jnp.sum(p, axis=1, keepdims=True)
        w_ref[...] = p[:, expert_col:expert_col + 1] / denom

    h = jax.lax.dot_general(
        x_ref[...], w1_ref[...], (((1,), (1,)), ((), ())),
        precision=precision, preferred_element_type=jnp.float32)
    h = h + b1_ref[...]
    h = 0.5 * h * (1.0 + jax.lax.erf(h * (1.0 / math.sqrt(2.0))))
    part = jax.lax.dot_general(
        h, w2_ref[...], (((1,), (1,)), ((), ())),
        precision=precision, preferred_element_type=jnp.float32)

    @pl.when(f == 0)
    def _init():
        out_ref[...] = part

    @pl.when(f != 0)
    def _acc():
        out_ref[...] += part

    @pl.when(f == n_ffn_tiles - 1)
    def _finish():
        out_ref[...] = (out_ref[...] + b2_ref[...]) * w_ref[...]


@functools.partial(jax.jit, static_argnames=())
def kernel(x, router_w, W1, b1, W2, b2):
    B_, S_, H_ = x.shape
    E_, FFN_, _ = W1.shape
    T = B_ * S_
    x_flat = x.reshape(T, H_)
    eid = E_ - 1
    w1 = W1[eid]
    b1e = b1[eid].reshape(1, FFN_)
    w2 = W2[eid]
    b2e = b2[eid].reshape(1, H_)

    TM = 2048
    FK = 512
    n_t = T // TM
    n_f = FFN_ // FK

    out = pl.pallas_call(
        functools.partial(_moe_kernel, n_ffn_tiles=n_f, expert_col=eid,
                          precision=jax.lax.Precision.DEFAULT),
        grid=(n_t, n_f),
        in_specs=[
            pl.BlockSpec((TM, H_), lambda t, f: (t, 0)),
            pl.BlockSpec((E_, H_), lambda t, f: (0, 0)),
            pl.BlockSpec((FK, H_), lambda t, f: (f, 0)),
            pl.BlockSpec((1, FK), lambda t, f: (0, f)),
            pl.BlockSpec((H_, FK), lambda t, f: (0, f)),
            pl.BlockSpec((1, H_), lambda t, f: (0, 0)),
        ],
        out_specs=pl.BlockSpec((TM, H_), lambda t, f: (t, 0)),
        out_shape=jax.ShapeDtypeStruct((T, H_), jnp.float32),
        scratch_shapes=[pltpu.VMEM((TM, 1), jnp.float32)],
        compiler_params=pltpu.CompilerParams(
            dimension_semantics=("parallel", "arbitrary")),
    )(x_flat, router_w, w1, b1e, w2, b2e)
    return out.reshape(B_, S_, H_)


# speedup vs baseline: 3.1531x; 1.1033x over previous
import functools
import math

import jax
import jax.numpy as jnp
from jax.experimental import pallas as pl
from jax.experimental.pallas import tpu as pltpu


def _moe_kernel(x_ref, rw_ref, w1_ref, b1_ref, w2_ref, b2_ref, out_ref, w_ref,
                *, n_ffn_tiles, expert_col):
    f = pl.program_id(1)

    @pl.when(f == 0)
    def _router():
        scores = jax.lax.dot_general(
            x_ref[...], rw_ref[...], (((1,), (1,)), ((), ())),
            preferred_element_type=jnp.float32)
        m = jnp.max(scores, axis=1, keepdims=True)
        p = jnp.exp(scores - m)
        denom = jnp.sum(p, axis=1, keepdims=True)
        w_ref[...] = p[:, expert_col:expert_col + 1] / denom

    h = jax.lax.dot_general(
        x_ref[...], w1_ref[...], (((1,), (1,)), ((), ())),
        preferred_element_type=jnp.float32)
    h = h + b1_ref[...]
    h = 0.5 * h * (1.0 + jax.lax.erf(h * (1.0 / math.sqrt(2.0))))
    part = jax.lax.dot_general(
        h.astype(jnp.bfloat16), w2_ref[...], (((1,), (1,)), ((), ())),
        preferred_element_type=jnp.float32)

    @pl.when(f == 0)
    def _init():
        out_ref[...] = part

    @pl.when(f != 0)
    def _acc():
        out_ref[...] += part

    @pl.when(f == n_ffn_tiles - 1)
    def _finish():
        out_ref[...] = (out_ref[...] + b2_ref[...]) * w_ref[...]


def kernel(x, router_w, W1, b1, W2, b2):
    B_, S_, H_ = x.shape
    E_, FFN_, _ = W1.shape
    T = B_ * S_
    eid = E_ - 1
    x_flat = x.reshape(T, H_).astype(jnp.bfloat16)
    rw = router_w.astype(jnp.bfloat16)
    w1 = W1[eid].astype(jnp.bfloat16)
    b1e = b1[eid].reshape(1, FFN_)
    w2 = W2[eid].astype(jnp.bfloat16)
    b2e = b2[eid].reshape(1, H_)

    TM = 2048
    FK = 1024
    n_t = T // TM
    n_f = FFN_ // FK

    out = pl.pallas_call(
        functools.partial(_moe_kernel, n_ffn_tiles=n_f, expert_col=eid),
        grid=(n_t, n_f),
        in_specs=[
            pl.BlockSpec((TM, H_), lambda t, f: (t, 0)),
            pl.BlockSpec((E_, H_), lambda t, f: (0, 0)),
            pl.BlockSpec((FK, H_), lambda t, f: (f, 0)),
            pl.BlockSpec((1, FK), lambda t, f: (0, f)),
            pl.BlockSpec((H_, FK), lambda t, f: (0, f)),
            pl.BlockSpec((1, H_), lambda t, f: (0, 0)),
        ],
        out_specs=pl.BlockSpec((TM, H_), lambda t, f: (t, 0)),
        out_shape=jax.ShapeDtypeStruct((T, H_), jnp.float32),
        scratch_shapes=[pltpu.VMEM((TM, 1), jnp.float32)],
        compiler_params=pltpu.CompilerParams(
            dimension_semantics=("parallel", "arbitrary")),
    )(x_flat, rw, w1, b1e, w2, b2e)
    return out.reshape(B_, S_, H_)
